# Initial kernel scaffold; baseline (speedup 1.0000x reference)
import jax
import jax.numpy as jnp
from jax.experimental import pallas as pl


def kernel(x, edge_index, W1, b1, W2, b2):
    raise NotImplementedError("write your pallas kernel here")


# speedup vs baseline: 3.5205x; 3.5205x over previous
import functools

import jax
import jax.numpy as jnp
from jax import lax
from jax.experimental import pallas as pl
from jax.experimental.pallas import tpu as pltpu
from jax.experimental.pallas import tpu_sc as plsc

NC = 2
NS = 16
C = 128


def _make_agg_kernel(n, d, ch, half, npad):
    mesh = plsc.VectorSubcoreMesh(core_axis_name="c", subcore_axis_name="s")
    init_rows = npad // NS
    out_rows = (half // NS) // 8 * 8
    rem_rows = half - out_rows * NS

    @functools.partial(
        pl.kernel,
        mesh=mesh,
        out_type=jax.ShapeDtypeStruct((n, d), jnp.float32),
        scratch_types=[
            pltpu.VMEM((ch, C), jnp.int32),
            pltpu.VMEM((ch, C), jnp.int32),
            pltpu.VMEM((C, d), jnp.float32),
            pltpu.VMEM((C, d), jnp.float32),
            pltpu.VMEM_SHARED((npad, d), jnp.float32),
            pltpu.SemaphoreType.DMA,
            pltpu.SemaphoreType.DMA,
        ],
    )
    def agg_kernel(x_hbm, src_hbm, dst_hbm, zeros_hbm, out_hbm,
                   src_v, dst_v, rows0, rows1, aggs, sem0, sem1):
        cid = lax.axis_index("c")
        sid = lax.axis_index("s")

        pltpu.sync_copy(src_hbm.at[sid], src_v)
        pltpu.sync_copy(dst_hbm.at[cid * NS + sid], dst_v)
        pltpu.sync_copy(zeros_hbm, aggs.at[pl.ds(sid * init_rows, init_rows)])
        plsc.subcore_barrier()

        def start(cj, rows, sem):
            pltpu.async_copy(x_hbm.at[src_v.at[cj]], rows, sem)

        def wait(cj, rows, sem):
            pltpu.make_async_copy(x_hbm.at[src_v.at[cj]], rows, sem).wait()

        def scat(cj, rows):
            pltpu.sync_copy(rows, aggs.at[dst_v.at[cj]], add=True)

        start(0, rows0, sem0)

        def body(j, carry):
            c0 = 2 * j
            start(c0 + 1, rows1, sem1)
            wait(c0, rows0, sem0)
            scat(c0, rows0)
            start(c0 + 2, rows0, sem0)
            wait(c0 + 1, rows1, sem1)
            scat(c0 + 1, rows1)
            return carry

        lax.fori_loop(0, ch // 2 - 1, body, 0)
        start(ch - 1, rows1, sem1)
        wait(ch - 2, rows0, sem0)
        scat(ch - 2, rows0)
        wait(ch - 1, rows1, sem1)
        scat(ch - 1, rows1)

        plsc.subcore_barrier()
        pltpu.sync_copy(
            aggs.at[pl.ds(sid * out_rows, out_rows)],
            out_hbm.at[pl.ds(cid * half + sid * out_rows, out_rows)],
        )
        if rem_rows:
            @pl.when(sid == 0)
            def _():
                pltpu.sync_copy(
                    aggs.at[pl.ds(NS * out_rows, rem_rows)],
                    out_hbm.at[pl.ds(cid * half + NS * out_rows, rem_rows)],
                )

    return agg_kernel


def _mlp(x, p, w1, b1, w2, b2):
    n, d = x.shape
    h = w2.shape[1]
    bn = 512

    def body(x_ref, p_ref, w1_ref, b1_ref, w2_ref, b2_ref, o_ref):
        acc = x_ref[...] + p_ref[...]
        acc = jnp.dot(acc, w1_ref[...], preferred_element_type=jnp.float32)
        acc = jnp.maximum(acc + b1_ref[...], 0.0)
        acc = jnp.dot(acc, w2_ref[...], preferred_element_type=jnp.float32)
        o_ref[...] = jnp.tanh(acc + b2_ref[...])

    return pl.pallas_call(
        body,
        grid=(pl.cdiv(n, bn),),
        in_specs=[
            pl.BlockSpec((bn, d), lambda i: (i, 0)),
            pl.BlockSpec((bn, d), lambda i: (i, 0)),
            pl.BlockSpec((d, h), lambda i: (0, 0)),
            pl.BlockSpec((1, h), lambda i: (0, 0)),
            pl.BlockSpec((h, h), lambda i: (0, 0)),
            pl.BlockSpec((1, h), lambda i: (0, 0)),
        ],
        out_specs=pl.BlockSpec((bn, h), lambda i: (i, 0)),
        out_shape=jax.ShapeDtypeStruct((n, h), jnp.float32),
    )(x, p, w1, b1.reshape(1, -1), w2, b2.reshape(1, -1))


def kernel(x, edge_index, W1, b1, W2, b2):
    n, d = x.shape
    e = edge_index.shape[1]
    half = -(-n // NC)

    ch = -(-e // (NS * C))
    ch += ch % 2
    total = NS * ch * C
    npad = -(-(half + 1) // NS) * NS

    src = jnp.concatenate([edge_index[0], jnp.zeros((total - e,), jnp.int32)])
    dst = jnp.concatenate([edge_index[1], jnp.full((total - e,), -1, jnp.int32)])
    src = src.reshape(NS, ch, C)
    lo = jnp.arange(NC, dtype=jnp.int32)[:, None] * half
    loc = dst[None, :] - lo
    loc = jnp.where((loc >= 0) & (loc < half), loc, half)
    dst2 = loc.reshape(NC * NS, ch, C)
    zeros = jnp.zeros((npad // NS, d), jnp.float32)

    agg = _make_agg_kernel(n, d, ch, half, npad)(x, src, dst2, zeros)
    return _mlp(x, agg, W1, b1, W2, b2)
